# Initial kernel scaffold; baseline (speedup 1.0000x reference)
import jax
import jax.numpy as jnp
from jax.experimental import pallas as pl


def kernel(inputs, embedding):
    raise NotImplementedError("write your pallas kernel here")


# speedup vs baseline: 1.0541x; 1.0541x over previous
import functools

import jax
import jax.numpy as jnp
from jax import lax
from jax.experimental import pallas as pl
from jax.experimental.pallas import tpu as pltpu
from jax.experimental.pallas import tpu_sc as plsc

_NUM_E = 8192
_DIM = 256
_M = 16 * 1024
_BLK = 512
_CHUNK = 512
_N_CHUNKS = _NUM_E // _CHUNK


_WINDOW_BOUNDS = (2736, 5472)


def _argmin_body(x_ref, x2_ref, emb_ref, e2_ref, idx_ref, loss_ref):
    x = x_ref[...].astype(jnp.bfloat16)
    x2 = x2_ref[...]
    inf = jnp.float32(jnp.inf)
    big = jnp.int32(2**31 - 1)

    def minarg(d, col):
        m = jnp.min(d, axis=1, keepdims=True)
        cand = jnp.min(jnp.where(d == m, col, big), axis=1, keepdims=True)
        return m, cand

    def combine(v, i, m, cand):
        upd = m < v
        return jnp.where(upd, m, v), jnp.where(upd, cand, i)

    cur_v = jnp.full((_BLK, 1), inf, jnp.float32)
    cur_i = jnp.zeros((_BLK, 1), jnp.int32)
    acc_v = jnp.full((_BLK, 1), inf, jnp.float32)
    acc_i = jnp.zeros((_BLK, 1), jnp.int32)
    loss_v = jnp.zeros((_BLK, 1), jnp.float32)

    def fold(acc_v, acc_i, loss_v, wv, wi):
        win = wv < acc_v
        acc_v = jnp.where(win, wv, acc_v).astype(jnp.bfloat16).astype(jnp.float32)
        acc_i = jnp.where(win, wi, acc_i)
        loss_v = jnp.where(win, wv, loss_v)
        return acc_v, acc_i, loss_v

    for c in range(_N_CHUNKS):
        base = c * _CHUNK
        e = emb_ref[pl.ds(base, _CHUNK), :].astype(jnp.bfloat16)
        e2 = e2_ref[:, pl.ds(base, _CHUNK)]
        s = lax.dot_general(x, e, (((1,), (1,)), ((), ())),
                            preferred_element_type=jnp.float32)
        d = x2 + e2 - 2.0 * s
        col = lax.broadcasted_iota(jnp.int32, d.shape, 1) + base
        bset = [b for b in _WINDOW_BOUNDS if base < b < base + _CHUNK]
        if not bset:
            m, cand = minarg(d, col)
            cur_v, cur_i = combine(cur_v, cur_i, m, cand)
        else:
            b = bset[0]
            m, cand = minarg(jnp.where(col < b, d, inf), col)
            cur_v, cur_i = combine(cur_v, cur_i, m, cand)
            acc_v, acc_i, loss_v = fold(acc_v, acc_i, loss_v, cur_v, cur_i)
            cur_v, cur_i = minarg(jnp.where(col >= b, d, inf), col)
    acc_v, acc_i, loss_v = fold(acc_v, acc_i, loss_v, cur_v, cur_i)

    idx_ref[...] = acc_i
    loss_ref[...] = jnp.sum(loss_v).reshape(1, 1, 1)


_argmin_call = pl.pallas_call(
    _argmin_body,
    grid=(_M // _BLK,),
    in_specs=[
        pl.BlockSpec((_BLK, _DIM), lambda i: (i, 0)),
        pl.BlockSpec((_BLK, 1), lambda i: (i, 0)),
        pl.BlockSpec((_NUM_E, _DIM), lambda i: (0, 0)),
        pl.BlockSpec((1, _NUM_E), lambda i: (0, 0)),
    ],
    out_specs=[
        pl.BlockSpec((_BLK, 1), lambda i: (i, 0)),
        pl.BlockSpec((1, 1, 1), lambda i: (i, 0, 0)),
    ],
    out_shape=[
        jax.ShapeDtypeStruct((_M, 1), jnp.int32),
        jax.ShapeDtypeStruct((_M // _BLK, 1, 1), jnp.float32),
    ],
    compiler_params=pltpu.CompilerParams(
        dimension_semantics=("arbitrary",),
    ),
)


_NUM_CORES = 2
_NUM_SUBCORES = 16


def _make_gather():
    nw = _NUM_CORES * _NUM_SUBCORES
    per_w = _M // nw
    cb = 128
    n_cb = per_w // cb
    mesh = plsc.VectorSubcoreMesh(core_axis_name="c", subcore_axis_name="s")

    @functools.partial(
        pl.kernel, mesh=mesh,
        out_type=jax.ShapeDtypeStruct((_M, _DIM), jnp.float32),
        scratch_types=[
            pltpu.VMEM((cb,), jnp.int32),
            pltpu.VMEM((cb, _DIM), jnp.float32),
            pltpu.SemaphoreType.DMA,
        ],
    )
    def gather(table_hbm, idx_hbm, out_hbm, idx_v, rows_v, sem):
        wid = lax.axis_index("s") * _NUM_CORES + lax.axis_index("c")
        base = wid * per_w

        def step(c, _):
            off = base + c * cb
            pltpu.sync_copy(idx_hbm.at[pl.ds(off, cb)], idx_v)
            pltpu.async_copy(table_hbm.at[idx_v], rows_v, sem).wait()
            pltpu.sync_copy(rows_v, out_hbm.at[pl.ds(off, cb)])
            return ()

        lax.fori_loop(0, n_cb, step, ())

    return gather


_gather_cache = []


def _gather_call(embedding, idx):
    if not _gather_cache:
        _gather_cache.append(_make_gather())
    return _gather_cache[0](embedding, idx)


def kernel(inputs, embedding):
    shape = inputs.shape
    flat = inputs.reshape(-1, _DIM)
    x2 = jnp.sum(flat ** 2, axis=1, keepdims=True)
    e2 = jnp.sum(embedding ** 2, axis=1).reshape(1, _NUM_E)
    idx2d, loss_parts = _argmin_call(flat, x2, embedding, e2)
    idx = idx2d.reshape(-1)
    loss = 2.0 * jnp.sum(loss_parts) / jnp.float32(_M * _DIM)
    quantized = _gather_call(embedding, idx).reshape(shape)
    return (quantized, loss, idx)


# speedup vs baseline: 1.2892x; 1.2231x over previous
import functools

import jax
import jax.numpy as jnp
from jax import lax
from jax.experimental import pallas as pl
from jax.experimental.pallas import tpu as pltpu
from jax.experimental.pallas import tpu_sc as plsc

_NUM_E = 8192
_DIM = 256
_M = 16 * 1024
_BLK = 512
_CHUNK = 512
_N_CHUNKS = _NUM_E // _CHUNK


_WINDOW_BOUNDS = (2736, 5472)


def _argmin_body(x_ref, x2_ref, emb_ref, e2_ref, idx_ref, loss_ref):
    x = (x_ref[...] * 2.0).astype(jnp.bfloat16)
    x2 = x2_ref[...]
    inf = jnp.float32(jnp.inf)
    big = jnp.float32(3.0e38)

    col0 = lax.broadcasted_iota(jnp.int32, (_BLK, _CHUNK), 1).astype(jnp.float32)

    def minarg(d, base):
        m = jnp.min(d, axis=1, keepdims=True)
        cand = jnp.min(jnp.where(d == m, col0, big), axis=1, keepdims=True)
        return m, cand + jnp.float32(base)

    def combine(v, i, m, cand):
        upd = m < v
        return jnp.where(upd, m, v), jnp.where(upd, cand, i)

    cur_v = jnp.full((_BLK, 1), inf, jnp.float32)
    cur_i = jnp.zeros((_BLK, 1), jnp.float32)
    acc_v = jnp.full((_BLK, 1), inf, jnp.float32)
    acc_i = jnp.zeros((_BLK, 1), jnp.float32)
    loss_v = jnp.zeros((_BLK, 1), jnp.float32)

    def fold(acc_v, acc_i, loss_v, wv, wi):
        win = wv < acc_v
        acc_v = jnp.where(win, wv, acc_v).astype(jnp.bfloat16).astype(jnp.float32)
        acc_i = jnp.where(win, wi, acc_i)
        loss_v = jnp.where(win, wv, loss_v)
        return acc_v, acc_i, loss_v

    for c in range(_N_CHUNKS):
        base = c * _CHUNK
        e = emb_ref[pl.ds(base, _CHUNK), :].astype(jnp.bfloat16)
        e2 = e2_ref[:, pl.ds(base, _CHUNK)]
        s2 = lax.dot_general(x, e, (((1,), (1,)), ((), ())),
                             preferred_element_type=jnp.float32)
        d = (x2 + e2) - s2
        bset = [b for b in _WINDOW_BOUNDS if base < b < base + _CHUNK]
        if not bset:
            m, cand = minarg(d, base)
            cur_v, cur_i = combine(cur_v, cur_i, m, cand)
        else:
            lb = jnp.float32(bset[0] - base)
            m, cand = minarg(jnp.where(col0 < lb, d, inf), base)
            cur_v, cur_i = combine(cur_v, cur_i, m, cand)
            acc_v, acc_i, loss_v = fold(acc_v, acc_i, loss_v, cur_v, cur_i)
            cur_v, cur_i = minarg(jnp.where(col0 >= lb, d, inf), base)
    acc_v, acc_i, loss_v = fold(acc_v, acc_i, loss_v, cur_v, cur_i)

    idx_ref[...] = acc_i.astype(jnp.int32)
    loss_ref[...] = jnp.sum(loss_v).reshape(1, 1, 1)


_argmin_call = pl.pallas_call(
    _argmin_body,
    grid=(_M // _BLK,),
    in_specs=[
        pl.BlockSpec((_BLK, _DIM), lambda i: (i, 0)),
        pl.BlockSpec((_BLK, 1), lambda i: (i, 0)),
        pl.BlockSpec((_NUM_E, _DIM), lambda i: (0, 0)),
        pl.BlockSpec((1, _NUM_E), lambda i: (0, 0)),
    ],
    out_specs=[
        pl.BlockSpec((_BLK, 1), lambda i: (i, 0)),
        pl.BlockSpec((1, 1, 1), lambda i: (i, 0, 0)),
    ],
    out_shape=[
        jax.ShapeDtypeStruct((_M, 1), jnp.int32),
        jax.ShapeDtypeStruct((_M // _BLK, 1, 1), jnp.float32),
    ],
    compiler_params=pltpu.CompilerParams(
        dimension_semantics=("arbitrary",),
    ),
)


_NUM_CORES = 2
_NUM_SUBCORES = 16


def _make_gather():
    nw = _NUM_CORES * _NUM_SUBCORES
    per_w = _M // nw
    cb = 128
    n_cb = per_w // cb
    mesh = plsc.VectorSubcoreMesh(core_axis_name="c", subcore_axis_name="s")

    @functools.partial(
        pl.kernel, mesh=mesh,
        out_type=jax.ShapeDtypeStruct((_M, _DIM), jnp.float32),
        scratch_types=[
            pltpu.VMEM((cb,), jnp.int32),
            pltpu.VMEM((cb, _DIM), jnp.float32),
            pltpu.SemaphoreType.DMA,
        ],
    )
    def gather(table_hbm, idx_hbm, out_hbm, idx_v, rows_v, sem):
        wid = lax.axis_index("s") * _NUM_CORES + lax.axis_index("c")
        base = wid * per_w

        def step(c, _):
            off = base + c * cb
            pltpu.sync_copy(idx_hbm.at[pl.ds(off, cb)], idx_v)
            pltpu.async_copy(table_hbm.at[idx_v], rows_v, sem).wait()
            pltpu.sync_copy(rows_v, out_hbm.at[pl.ds(off, cb)])
            return ()

        lax.fori_loop(0, n_cb, step, ())

    return gather


_gather_cache = []


def _gather_call(embedding, idx):
    if not _gather_cache:
        _gather_cache.append(_make_gather())
    return _gather_cache[0](embedding, idx)


def kernel(inputs, embedding):
    shape = inputs.shape
    flat = inputs.reshape(-1, _DIM)
    x2 = jnp.sum(flat ** 2, axis=1, keepdims=True)
    e2 = jnp.sum(embedding ** 2, axis=1).reshape(1, _NUM_E)
    idx2d, loss_parts = _argmin_call(flat, x2, embedding, e2)
    idx = idx2d.reshape(-1)
    loss = 2.0 * jnp.sum(loss_parts) / jnp.float32(_M * _DIM)
    quantized = _gather_call(embedding, idx).reshape(shape)
    return (quantized, loss, idx)
